# baseline (speedup 1.0000x reference)
import jax
import jax.numpy as jnp
from jax import lax
from jax.experimental import pallas as pl
from jax.experimental.pallas import tpu as pltpu
from jax.experimental.pallas import tpu_sc as plsc

_N = 10000
_E = 160000
_D = 256
_DH = 128
_NC = 2
_NS = 16
_CH = 128
_NCK = 80
_NCKH = 16
_NPART = _NCK // _NCKH
_EPAD = _NCK * _CH * _NS
_NPAD = 10240
_DUMMY = _N
_ZCH = _NPAD // _NS // _CH
_FLUSH = _NPAD // _NS
_BN = 1000


def _sc_body(xcat, src2, dst3, z128, z1d, s_out, c_out,
             acc, src_v, dst_v, rows_a, rows_b, hist,
             gsem_a, gsem_b, ssem_a, ssem_b):
    c = lax.axis_index("c")
    s = lax.axis_index("s")
    ones16 = jnp.ones((16,), jnp.float32)
    rows = (rows_a, rows_b)
    gsems = (gsem_a, gsem_b)
    ssems = (ssem_a, ssem_b)
    for et in range(3):
        pltpu.sync_copy(z128, acc.at[pl.ds(s * _FLUSH, _FLUSH)])
        pltpu.sync_copy(z1d, hist)
        plsc.subcore_barrier()

        @pl.loop(0, _NPART)
        def _part(p):
            base = s * _NCK + p * _NCKH
            pltpu.sync_copy(src2.at[c, et, pl.ds(base, _NCKH), :], src_v)
            pltpu.sync_copy(dst3.at[et, pl.ds(base, _NCKH), :], dst_v)
            gd = [None, None]
            sd = [None, None]
            gd[0] = pltpu.async_copy(xcat.at[src_v.at[0]], rows[0], gsems[0])
            for j in range(_NCKH):
                b, nb = j % 2, (j + 1) % 2
                if j + 1 < _NCKH:
                    if sd[nb] is not None:
                        sd[nb].wait()
                    gd[nb] = pltpu.async_copy(
                        xcat.at[src_v.at[j + 1]], rows[nb], gsems[nb])
                gd[b].wait()
                sd[b] = pltpu.async_copy(
                    rows[b], acc.at[dst_v.at[j]], ssems[b], add=True)

                goff = c * (_CH // 32)
                for g in range(_CH // 32):
                    idx16 = dst_v[j, pl.ds((goff + g) * 16, 16)]
                    plsc.addupdate_scatter(hist, [idx16], ones16)
            sd[0].wait()
            sd[1].wait()

        plsc.subcore_barrier()
        pltpu.sync_copy(acc.at[pl.ds(s * _FLUSH, _FLUSH)],
                        s_out.at[c, et, pl.ds(s * _FLUSH, _FLUSH), :])
        pltpu.sync_copy(hist, c_out.at[c, et, s])
        plsc.subcore_barrier()


def _sc_segment_sums(xcat, src2, dst3, z128, z1d):
    fn = pl.kernel(
        _sc_body,
        out_type=(jax.ShapeDtypeStruct((_NC, 3, _NPAD, _DH), jnp.float32),
                  jax.ShapeDtypeStruct((_NC, 3, _NS, _NPAD), jnp.float32)),
        mesh=plsc.VectorSubcoreMesh(core_axis_name="c", subcore_axis_name="s"),
        compiler_params=pltpu.CompilerParams(needs_layout_passes=False),
        scratch_types=[
            pltpu.VMEM_SHARED((_NPAD, _DH), jnp.float32),
            pltpu.VMEM((_NCKH, _CH), jnp.int32),
            pltpu.VMEM((_NCKH, _CH), jnp.int32),
            pltpu.VMEM((_CH, _DH), jnp.float32),
            pltpu.VMEM((_CH, _DH), jnp.float32),
            pltpu.VMEM((_NPAD,), jnp.float32),
            pltpu.SemaphoreType.DMA,
            pltpu.SemaphoreType.DMA,
            pltpu.SemaphoreType.DMA,
            pltpu.SemaphoreType.DMA,
        ],
    )
    return fn(xcat, src2, dst3, z128, z1d)


def _tc_body(s_ref, c_ref, w_ref, b_ref, h_ref):
    h = jnp.zeros((_BN, _D), jnp.float32)
    for et in range(3):
        cnt = jnp.sum(c_ref[et], axis=1, keepdims=True)
        recip = 1.0 / jnp.maximum(cnt, 1.0)
        for c in range(_NC):
            h = h + jnp.dot(s_ref[c, et] * recip, w_ref[c, et],
                            preferred_element_type=jnp.float32)
        h = h + (cnt > 0).astype(jnp.float32) * b_ref[et][None, :]
    h_ref[...] = h


def _tc_combine(s_all, c_all, w_split, b_stack):
    return pl.pallas_call(
        _tc_body,
        grid=(_N // _BN,),
        in_specs=[
            pl.BlockSpec((_NC, 3, _BN, _DH), lambda i: (0, 0, i, 0)),
            pl.BlockSpec((3, _BN, 2 * _NS), lambda i: (0, i, 0)),
            pl.BlockSpec((_NC, 3, _DH, _D), lambda i: (0, 0, 0, 0)),
            pl.BlockSpec((3, _D), lambda i: (0, 0)),
        ],
        out_specs=pl.BlockSpec((_BN, _D), lambda i: (i, 0)),
        out_shape=jax.ShapeDtypeStruct((_N, _D), jnp.float32),
    )(s_all, c_all, w_split, b_stack)


def _pad_edges(edge_index):
    src = jnp.concatenate(
        [edge_index[0], jnp.zeros((_EPAD - _E,), jnp.int32)])
    dst = jnp.concatenate(
        [edge_index[1], jnp.full((_EPAD - _E,), _DUMMY, jnp.int32)])
    return src.reshape(-1, _CH), dst.reshape(-1, _CH)


@jax.jit
def kernel(x, edge_index_follows, edge_index_likes, edge_index_writes,
           W_follows, b_follows, W_likes, b_likes, W_writes, b_writes):
    xcat = jnp.concatenate([x[:, :_DH], x[:, _DH:]], axis=0)
    pairs = [_pad_edges(e) for e in
             (edge_index_follows, edge_index_likes, edge_index_writes)]
    src_all = jnp.stack([p[0] for p in pairs])
    dst3 = jnp.stack([p[1] for p in pairs])
    src2 = jnp.stack([src_all, src_all + _N])
    z128 = jnp.zeros((_FLUSH, _DH), jnp.float32)
    z1d = jnp.zeros((_NPAD,), jnp.float32)

    s_all, c_parts = _sc_segment_sums(xcat, src2, dst3, z128, z1d)
    s_all = s_all[:, :, :_N, :]
    c_all = jnp.transpose(c_parts, (1, 3, 0, 2)).reshape(3, _NPAD, 32)[:, :_N, :]

    ws = [W_follows, W_likes, W_writes]
    w_split = jnp.stack([jnp.stack([w[:_DH, :] for w in ws]),
                         jnp.stack([w[_DH:, :] for w in ws])])
    b_stack = jnp.stack([b_follows, b_likes, b_writes])
    return _tc_combine(s_all, c_all, w_split, b_stack)


# speedup vs baseline: 1.0143x; 1.0143x over previous
import jax
import jax.numpy as jnp
from jax import lax
from jax.experimental import pallas as pl
from jax.experimental.pallas import tpu as pltpu
from jax.experimental.pallas import tpu_sc as plsc

_N = 10000
_E = 160000
_D = 256
_DH = 128
_NC = 2
_NS = 16
_CH = 128
_NCK = 80
_NCKH = 16
_NPART = _NCK // _NCKH
_EPAD = _NCK * _CH * _NS
_NPAD = 10240
_DUMMY = _N
_ZCH = _NPAD // _NS // _CH
_FLUSH = _NPAD // _NS
_BN = 1000


def _sc_body(xcat, src2, dst3, z128, z1d, s_out, c_out,
             acc, src_v, dst_v, rows_a, rows_b, hist,
             gsem_a, gsem_b, ssem_a, ssem_b):
    c = lax.axis_index("c")
    s = lax.axis_index("s")
    ones16 = jnp.ones((16,), jnp.float32)
    rows = (rows_a, rows_b)
    gsems = (gsem_a, gsem_b)
    ssems = (ssem_a, ssem_b)
    for et in range(3):
        pltpu.sync_copy(z128, acc.at[pl.ds(s * _FLUSH, _FLUSH)])
        pltpu.sync_copy(z1d, hist)
        plsc.subcore_barrier()

        @pl.loop(0, _NPART)
        def _part(p):
            base = s * _NCK + p * _NCKH
            pltpu.sync_copy(src2.at[c, et, pl.ds(base, _NCKH), :], src_v)
            pltpu.sync_copy(dst3.at[et, pl.ds(base, _NCKH), :], dst_v)
            gd = [None, None]
            sd = [None, None]
            gd[0] = pltpu.async_copy(xcat.at[src_v.at[0]], rows[0], gsems[0])
            for j in range(_NCKH):
                b, nb = j % 2, (j + 1) % 2
                if j + 1 < _NCKH:
                    if sd[nb] is not None:
                        sd[nb].wait()
                    gd[nb] = pltpu.async_copy(
                        xcat.at[src_v.at[j + 1]], rows[nb], gsems[nb])
                gd[b].wait()
                sd[b] = pltpu.async_copy(
                    rows[b], acc.at[dst_v.at[j]], ssems[b], add=True)

                @pl.when(c == 0)
                def _():
                    for g in range(_CH // 16):
                        idx16 = dst_v[j, pl.ds(g * 16, 16)]
                        plsc.addupdate_scatter(hist, [idx16], ones16)
            sd[0].wait()
            sd[1].wait()

        plsc.subcore_barrier()
        pltpu.sync_copy(acc.at[pl.ds(s * _FLUSH, _FLUSH)],
                        s_out.at[c, et, pl.ds(s * _FLUSH, _FLUSH), :])
        pltpu.sync_copy(hist, c_out.at[c, et, s])
        plsc.subcore_barrier()


def _sc_segment_sums(xcat, src2, dst3, z128, z1d):
    fn = pl.kernel(
        _sc_body,
        out_type=(jax.ShapeDtypeStruct((_NC, 3, _NPAD, _DH), jnp.float32),
                  jax.ShapeDtypeStruct((_NC, 3, _NS, _NPAD), jnp.float32)),
        mesh=plsc.VectorSubcoreMesh(core_axis_name="c", subcore_axis_name="s"),
        compiler_params=pltpu.CompilerParams(needs_layout_passes=False,
                                             use_tc_tiling_on_sc=False),
        scratch_types=[
            pltpu.VMEM_SHARED((_NPAD, _DH), jnp.float32),
            pltpu.VMEM((_NCKH, _CH), jnp.int32),
            pltpu.VMEM((_NCKH, _CH), jnp.int32),
            pltpu.VMEM((_CH, _DH), jnp.float32),
            pltpu.VMEM((_CH, _DH), jnp.float32),
            pltpu.VMEM((_NPAD,), jnp.float32),
            pltpu.SemaphoreType.DMA,
            pltpu.SemaphoreType.DMA,
            pltpu.SemaphoreType.DMA,
            pltpu.SemaphoreType.DMA,
        ],
    )
    return fn(xcat, src2, dst3, z128, z1d)


def _tc_body(s_ref, c_ref, w_ref, b_ref, h_ref):
    h = jnp.zeros((_BN, _D), jnp.float32)
    for et in range(3):
        cnt = jnp.sum(c_ref[et], axis=1, keepdims=True)
        recip = 1.0 / jnp.maximum(cnt, 1.0)
        for c in range(_NC):
            h = h + jnp.dot(s_ref[c, et] * recip, w_ref[c, et],
                            preferred_element_type=jnp.float32)
        h = h + (cnt > 0).astype(jnp.float32) * b_ref[et][None, :]
    h_ref[...] = h


def _tc_combine(s_all, c_all, w_split, b_stack):
    return pl.pallas_call(
        _tc_body,
        grid=(_N // _BN,),
        in_specs=[
            pl.BlockSpec((_NC, 3, _BN, _DH), lambda i: (0, 0, i, 0)),
            pl.BlockSpec((3, _BN, _NS), lambda i: (0, i, 0)),
            pl.BlockSpec((_NC, 3, _DH, _D), lambda i: (0, 0, 0, 0)),
            pl.BlockSpec((3, _D), lambda i: (0, 0)),
        ],
        out_specs=pl.BlockSpec((_BN, _D), lambda i: (i, 0)),
        out_shape=jax.ShapeDtypeStruct((_N, _D), jnp.float32),
    )(s_all, c_all, w_split, b_stack)


def _pad_edges(edge_index):
    src = jnp.concatenate(
        [edge_index[0], jnp.zeros((_EPAD - _E,), jnp.int32)])
    dst = jnp.concatenate(
        [edge_index[1], jnp.full((_EPAD - _E,), _DUMMY, jnp.int32)])
    return src.reshape(-1, _CH), dst.reshape(-1, _CH)


@jax.jit
def kernel(x, edge_index_follows, edge_index_likes, edge_index_writes,
           W_follows, b_follows, W_likes, b_likes, W_writes, b_writes):
    xcat = jnp.concatenate([x[:, :_DH], x[:, _DH:]], axis=0)
    pairs = [_pad_edges(e) for e in
             (edge_index_follows, edge_index_likes, edge_index_writes)]
    src_all = jnp.stack([p[0] for p in pairs])
    dst3 = jnp.stack([p[1] for p in pairs])
    src2 = jnp.stack([src_all, src_all + _N])
    z128 = jnp.zeros((_FLUSH, _DH), jnp.float32)
    z1d = jnp.zeros((_NPAD,), jnp.float32)

    s_all, c_parts = _sc_segment_sums(xcat, src2, dst3, z128, z1d)
    s_all = s_all[:, :, :_N, :]
    c_all = jnp.transpose(c_parts[0], (0, 2, 1))[:, :_N, :]

    ws = [W_follows, W_likes, W_writes]
    w_split = jnp.stack([jnp.stack([w[:_DH, :] for w in ws]),
                         jnp.stack([w[_DH:, :] for w in ws])])
    b_stack = jnp.stack([b_follows, b_likes, b_writes])
    return _tc_combine(s_all, c_all, w_split, b_stack)


# speedup vs baseline: 1.0533x; 1.0385x over previous
import jax
import jax.numpy as jnp
from jax import lax
from jax.experimental import pallas as pl
from jax.experimental.pallas import tpu as pltpu
from jax.experimental.pallas import tpu_sc as plsc

_N = 10000
_E = 160000
_D = 256
_DH = 128
_NC = 2
_NS = 16
_CH = 128
_NCK = 80
_NCKH = 16
_NPART = _NCK // _NCKH
_EPAD = _NCK * _CH * _NS
_NPAD = 10240
_DUMMY = _N
_ZCH = _NPAD // _NS // _CH
_FLUSH = _NPAD // _NS
_BN = 1000


def _sc_body(xcat, src2, dst3, z128, z1d, s_out, c_out,
             acc, src_v, dst_v, rows_a, rows_b, hist,
             gsem_a, gsem_b, ssem_a, ssem_b):
    c = lax.axis_index("c")
    s = lax.axis_index("s")
    ones16 = jnp.ones((16,), jnp.float32)
    rows = (rows_a, rows_b)
    gsems = (gsem_a, gsem_b)
    ssems = (ssem_a, ssem_b)
    for et in range(3):
        pltpu.sync_copy(z128, acc.at[pl.ds(s * _FLUSH, _FLUSH)])
        pltpu.sync_copy(z1d, hist)
        plsc.subcore_barrier()

        @pl.loop(0, _NPART)
        def _part(p):
            base = s * _NCK + p * _NCKH
            pltpu.sync_copy(src2.at[c, et, pl.ds(base, _NCKH), :], src_v)
            pltpu.sync_copy(dst3.at[et, pl.ds(base, _NCKH), :], dst_v)
            gd = [None, None]
            sd = [None, None]
            gd[0] = pltpu.async_copy(xcat.at[src_v.at[0]], rows[0], gsems[0])
            for j in range(_NCKH):
                b, nb = j % 2, (j + 1) % 2
                if j + 1 < _NCKH:
                    if sd[nb] is not None:
                        sd[nb].wait()
                    gd[nb] = pltpu.async_copy(
                        xcat.at[src_v.at[j + 1]], rows[nb], gsems[nb])
                gd[b].wait()
                sd[b] = pltpu.async_copy(
                    rows[b], acc.at[dst_v.at[j]], ssems[b], add=True)

                @pl.when(c == 0)
                def _():
                    for g in range(_CH // 16):
                        idx16 = dst_v[j, pl.ds(g * 16, 16)]
                        plsc.addupdate_scatter(hist, [idx16], ones16)
            sd[0].wait()
            sd[1].wait()

        plsc.subcore_barrier()
        pltpu.sync_copy(acc.at[pl.ds(s * _FLUSH, _FLUSH)],
                        s_out.at[c, et, pl.ds(s * _FLUSH, _FLUSH), :])
        pltpu.sync_copy(hist, c_out.at[c, et, s])
        plsc.subcore_barrier()


def _sc_segment_sums(xcat, src2, dst3, z128, z1d):
    fn = pl.kernel(
        _sc_body,
        out_type=(jax.ShapeDtypeStruct((_NC, 3, _NPAD, _DH), jnp.float32),
                  jax.ShapeDtypeStruct((_NC, 3, _NS, _NPAD), jnp.float32)),
        mesh=plsc.VectorSubcoreMesh(core_axis_name="c", subcore_axis_name="s"),
        compiler_params=pltpu.CompilerParams(needs_layout_passes=False),
        scratch_types=[
            pltpu.VMEM_SHARED((_NPAD, _DH), jnp.float32),
            pltpu.VMEM((_NCKH, _CH), jnp.int32),
            pltpu.VMEM((_NCKH, _CH), jnp.int32),
            pltpu.VMEM((_CH, _DH), jnp.float32),
            pltpu.VMEM((_CH, _DH), jnp.float32),
            pltpu.VMEM((_NPAD,), jnp.float32),
            pltpu.SemaphoreType.DMA,
            pltpu.SemaphoreType.DMA,
            pltpu.SemaphoreType.DMA,
            pltpu.SemaphoreType.DMA,
        ],
    )
    return fn(xcat, src2, dst3, z128, z1d)


def _tc_body(s_ref, c_ref, w_ref, b_ref, h_ref):
    h = jnp.zeros((_BN, _D), jnp.float32)
    for et in range(3):
        cnt = jnp.sum(c_ref[et], axis=1, keepdims=True)
        recip = 1.0 / jnp.maximum(cnt, 1.0)
        for c in range(_NC):
            h = h + jnp.dot(s_ref[c, et] * recip, w_ref[c, et],
                            preferred_element_type=jnp.float32)
        h = h + (cnt > 0).astype(jnp.float32) * b_ref[et][None, :]
    h_ref[...] = h


def _tc_combine(s_all, c_all, w_split, b_stack):
    return pl.pallas_call(
        _tc_body,
        grid=(_N // _BN,),
        in_specs=[
            pl.BlockSpec((_NC, 3, _BN, _DH), lambda i: (0, 0, i, 0)),
            pl.BlockSpec((3, _BN, _NS), lambda i: (0, i, 0)),
            pl.BlockSpec((_NC, 3, _DH, _D), lambda i: (0, 0, 0, 0)),
            pl.BlockSpec((3, _D), lambda i: (0, 0)),
        ],
        out_specs=pl.BlockSpec((_BN, _D), lambda i: (i, 0)),
        out_shape=jax.ShapeDtypeStruct((_N, _D), jnp.float32),
    )(s_all, c_all, w_split, b_stack)


def _pad_edges(edge_index):
    src = jnp.concatenate(
        [edge_index[0], jnp.zeros((_EPAD - _E,), jnp.int32)])
    dst = jnp.concatenate(
        [edge_index[1], jnp.full((_EPAD - _E,), _DUMMY, jnp.int32)])
    return src.reshape(-1, _CH), dst.reshape(-1, _CH)


@jax.jit
def kernel(x, edge_index_follows, edge_index_likes, edge_index_writes,
           W_follows, b_follows, W_likes, b_likes, W_writes, b_writes):
    xcat = jnp.concatenate([x[:, :_DH], x[:, _DH:]], axis=0)
    pairs = [_pad_edges(e) for e in
             (edge_index_follows, edge_index_likes, edge_index_writes)]
    src_all = jnp.stack([p[0] for p in pairs])
    dst3 = jnp.stack([p[1] for p in pairs])
    src2 = jnp.stack([src_all, src_all + _N])
    z128 = jnp.zeros((_FLUSH, _DH), jnp.float32)
    z1d = jnp.zeros((_NPAD,), jnp.float32)

    s_all, c_parts = _sc_segment_sums(xcat, src2, dst3, z128, z1d)
    s_all = s_all[:, :, :_N, :]
    c_all = jnp.transpose(c_parts[0], (0, 2, 1))[:, :_N, :]

    ws = [W_follows, W_likes, W_writes]
    w_split = jnp.stack([jnp.stack([w[:_DH, :] for w in ws]),
                         jnp.stack([w[_DH:, :] for w in ws])])
    b_stack = jnp.stack([b_follows, b_likes, b_writes])
    return _tc_combine(s_all, c_all, w_split, b_stack)
